# baseline (speedup 1.0000x reference)
import functools

import jax
import jax.numpy as jnp
from jax import lax
from jax.experimental import pallas as pl
from jax.experimental.pallas import tpu as pltpu
from jax.experimental.pallas import tpu_sc as plsc

N = 10000
E = 320000
D = 128
H = 128
OUT = 64
G = 64
BN_EPS = 1e-5

NC = 2
NS = 16
NW = NC * NS
CHUNK = 80
EPW = E // NW
NCHUNK = EPW // CHUNK
PHASES = ((0, 32), (32, 32), (64, 32), (96, 29))
STAGE = 32
RPT = 624
TAIL = N - NS * RPT


def _sc_segment_sum(h, src2d, dst2d, zeros):
    mesh = plsc.VectorSubcoreMesh(core_axis_name="c", subcore_axis_name="s")

    @functools.partial(
        pl.kernel,
        out_type=jax.ShapeDtypeStruct((NC, N, D), jnp.float32),
        mesh=mesh,
        scratch_types=[
            pltpu.VMEM((STAGE, CHUNK), jnp.int32),
            pltpu.VMEM((STAGE, CHUNK), jnp.int32),
            pltpu.VMEM((CHUNK, D), jnp.float32),
            pltpu.VMEM((CHUNK, D), jnp.float32),
            pltpu.VMEM((CHUNK, D), jnp.float32),
            pltpu.VMEM_SHARED((N, D), jnp.float32),
            pltpu.SemaphoreType.DMA,
            pltpu.SemaphoreType.DMA,
            pltpu.SemaphoreType.DMA,
        ],
    )
    def k(h_hbm, src_hbm, dst_hbm, z_hbm, out_hbm,
          sidx, didx, rows0, rows1, rows2, acc, sem0, sem1, sem2):
        cid = lax.axis_index("c")
        sid = lax.axis_index("s")
        wid = cid * NS + sid

        pltpu.sync_copy(
            z_hbm.at[pl.ds(sid * RPT, RPT)],
            acc.at[pl.ds(sid * RPT, RPT)],
        )

        @pl.when(sid == NS - 1)
        def _():
            pltpu.sync_copy(
                z_hbm.at[pl.ds(NS * RPT, TAIL)],
                acc.at[pl.ds(NS * RPT, TAIL)],
            )
        plsc.subcore_barrier()

        def start_gather(c, buf, sem):
            pltpu.async_copy(h_hbm.at[sidx.at[c]], buf, sem)

        def wait_gather(buf, sem):
            pltpu.make_async_copy(h_hbm.at[sidx.at[0]], buf, sem).wait()

        def scatter(c, buf):
            pltpu.sync_copy(buf, acc.at[didx.at[c]], add=True)

        def do_phase(start, cnt):
            assert cnt % 3 == 2 and cnt >= 2
            pltpu.sync_copy(src_hbm.at[wid].at[pl.ds(start, cnt)],
                            sidx.at[pl.ds(0, cnt)])
            pltpu.sync_copy(dst_hbm.at[wid].at[pl.ds(start, cnt)],
                            didx.at[pl.ds(0, cnt)])
            start_gather(0, rows0, sem0)
            start_gather(1, rows1, sem1)

            @pl.loop(0, (cnt - 2) // 3)
            def _(k3):
                c0 = 3 * k3
                start_gather(c0 + 2, rows2, sem2)
                wait_gather(rows0, sem0)
                scatter(c0, rows0)
                start_gather(c0 + 3, rows0, sem0)
                wait_gather(rows1, sem1)
                scatter(c0 + 1, rows1)
                start_gather(c0 + 4, rows1, sem1)
                wait_gather(rows2, sem2)
                scatter(c0 + 2, rows2)

            wait_gather(rows0, sem0)
            scatter(cnt - 2, rows0)
            wait_gather(rows1, sem1)
            scatter(cnt - 1, rows1)

        for start, cnt in PHASES:
            do_phase(start, cnt)

        plsc.subcore_barrier()
        pltpu.sync_copy(
            acc.at[pl.ds(sid * RPT, RPT)],
            out_hbm.at[cid].at[pl.ds(sid * RPT, RPT)],
        )

        @pl.when(sid == NS - 1)
        def _():
            pltpu.sync_copy(
                acc.at[pl.ds(NS * RPT, TAIL)],
                out_hbm.at[cid].at[pl.ds(NS * RPT, TAIL)],
            )

    return k(h, src2d, dst2d, zeros)


def _bn_relu(z, g, b):
    mu = jnp.mean(z, axis=0, keepdims=True)
    zc = z - mu
    var = jnp.mean(zc * zc, axis=0, keepdims=True)
    z = zc * lax.rsqrt(var + BN_EPS) * g + b
    return jnp.maximum(z, 0.0)


def _tc_layer(h, p, W1, b1, g1, be1, W2, b2, gout, bout):
    def body(h_ref, p_ref, W1_ref, b1_ref, g1_ref, be1_ref, W2_ref, b2_ref,
             gout_ref, bout_ref, o_ref):
        z = h_ref[...] + p_ref[0] + p_ref[1]
        z = jnp.dot(z, W1_ref[...], preferred_element_type=jnp.float32) + b1_ref[...]
        z = _bn_relu(z, g1_ref[...], be1_ref[...])
        z = jnp.dot(z, W2_ref[...], preferred_element_type=jnp.float32) + b2_ref[...]
        o_ref[...] = _bn_relu(z, gout_ref[...], bout_ref[...])

    return pl.pallas_call(
        body,
        out_shape=jax.ShapeDtypeStruct((N, H), jnp.float32),
    )(h, p, W1, b1, g1, be1, W2, b2, gout, bout)


def _tc_layer_pool(h, p, W1, b1, g1, be1, W2, b2, gout, bout,
                   batch2d, pred_W, pred_b):
    def body(h_ref, p_ref, W1_ref, b1_ref, g1_ref, be1_ref, W2_ref, b2_ref,
             gout_ref, bout_ref, batch_ref, pW_ref, pb_ref, o_ref):
        z = h_ref[...] + p_ref[0] + p_ref[1]
        z = jnp.dot(z, W1_ref[...], preferred_element_type=jnp.float32) + b1_ref[...]
        z = _bn_relu(z, g1_ref[...], be1_ref[...])
        z = jnp.dot(z, W2_ref[...], preferred_element_type=jnp.float32) + b2_ref[...]
        h2 = _bn_relu(z, gout_ref[...], bout_ref[...])

        gids = lax.broadcasted_iota(jnp.int32, (G, N), 0)
        oh = (batch_ref[...] == gids).astype(jnp.float32)
        sums = jnp.dot(oh, h2, preferred_element_type=jnp.float32)
        counts = jnp.sum(oh, axis=1, keepdims=True)
        pooled = sums / jnp.maximum(counts, 1.0)
        out = jnp.dot(pooled, pW_ref[...], preferred_element_type=jnp.float32)
        out = out + pb_ref[...]
        m = jnp.max(out, axis=-1, keepdims=True)
        sh = out - m
        lse = jnp.log(jnp.sum(jnp.exp(sh), axis=-1, keepdims=True))
        o_ref[...] = sh - lse

    return pl.pallas_call(
        body,
        out_shape=jax.ShapeDtypeStruct((G, OUT), jnp.float32),
    )(h, p, W1, b1, g1, be1, W2, b2, gout, bout, batch2d, pred_W, pred_b)


def kernel(x, params, edge_index, batch):
    src2d = edge_index[0].reshape(NW, NCHUNK, CHUNK)
    dst2d = edge_index[1].reshape(NW, NCHUNK, CHUNK)
    zeros = jnp.zeros((N, D), jnp.float32)
    batch2d = batch.reshape(1, N)

    l0, l1 = params["layers"]
    r = lambda v: v.reshape(1, -1)

    p1 = _sc_segment_sum(x, src2d, dst2d, zeros)
    h1 = _tc_layer(x, p1, l0["W1"], r(l0["b1"]), r(l0["g1"]), r(l0["be1"]),
                   l0["W2"], r(l0["b2"]), r(l0["gout"]), r(l0["bout"]))
    p2 = _sc_segment_sum(h1, src2d, dst2d, zeros)
    return _tc_layer_pool(h1, p2, l1["W1"], r(l1["b1"]), r(l1["g1"]),
                          r(l1["be1"]), l1["W2"], r(l1["b2"]), r(l1["gout"]),
                          r(l1["bout"]), batch2d, params["pred_W"],
                          r(params["pred_b"]))


# speedup vs baseline: 1.0546x; 1.0546x over previous
import functools

import jax
import jax.numpy as jnp
from jax import lax
from jax.experimental import pallas as pl
from jax.experimental.pallas import tpu as pltpu
from jax.experimental.pallas import tpu_sc as plsc

N = 10000
E = 320000
D = 128
H = 128
OUT = 64
G = 64
BN_EPS = 1e-5

NC = 2
NS = 16
NW = NC * NS
CHUNK = 80
EPW = E // NW
NCHUNK = EPW // CHUNK
PHASES = ((0, 32), (32, 32), (64, 32), (96, 29))
STAGE = 32
RPT = 624
TAIL = N - NS * RPT


def _sc_segment_sum(h, edges4d, zeros):
    mesh = plsc.VectorSubcoreMesh(core_axis_name="c", subcore_axis_name="s")

    @functools.partial(
        pl.kernel,
        out_type=jax.ShapeDtypeStruct((NC, N, D), jnp.float32),
        mesh=mesh,
        scratch_types=[
            pltpu.VMEM((STAGE, CHUNK), jnp.int32),
            pltpu.VMEM((STAGE, CHUNK), jnp.int32),
            pltpu.VMEM((CHUNK, D), jnp.float32),
            pltpu.VMEM((CHUNK, D), jnp.float32),
            pltpu.VMEM((CHUNK, D), jnp.float32),
            pltpu.VMEM_SHARED((N, D), jnp.float32),
            pltpu.SemaphoreType.DMA,
            pltpu.SemaphoreType.DMA,
            pltpu.SemaphoreType.DMA,
        ],
    )
    def k(h_hbm, e_hbm, z_hbm, out_hbm,
          sidx, didx, rows0, rows1, rows2, acc, sem0, sem1, sem2):
        cid = lax.axis_index("c")
        sid = lax.axis_index("s")
        wid = cid * NS + sid
        src_hbm = e_hbm.at[0]
        dst_hbm = e_hbm.at[1]

        pltpu.async_copy(
            z_hbm.at[pl.ds(sid * RPT, RPT)],
            acc.at[pl.ds(sid * RPT, RPT)],
            sem2,
        )

        @pl.when(sid == NS - 1)
        def _():
            pltpu.sync_copy(
                z_hbm.at[pl.ds(NS * RPT, TAIL)],
                acc.at[pl.ds(NS * RPT, TAIL)],
            )

        def start_gather(c, buf, sem):
            pltpu.async_copy(h_hbm.at[sidx.at[c]], buf, sem)

        def wait_gather(buf, sem):
            pltpu.make_async_copy(h_hbm.at[sidx.at[0]], buf, sem).wait()

        def scatter(c, buf):
            pltpu.sync_copy(buf, acc.at[didx.at[c]], add=True)

        def do_phase(start, cnt):
            assert cnt % 3 == 2 and cnt >= 2
            pltpu.sync_copy(src_hbm.at[wid].at[pl.ds(start, cnt)],
                            sidx.at[pl.ds(0, cnt)])
            pltpu.sync_copy(dst_hbm.at[wid].at[pl.ds(start, cnt)],
                            didx.at[pl.ds(0, cnt)])
            if start == 0:
                pltpu.make_async_copy(
                    z_hbm.at[pl.ds(sid * RPT, RPT)],
                    acc.at[pl.ds(sid * RPT, RPT)],
                    sem2,
                ).wait()
                plsc.subcore_barrier()
            start_gather(0, rows0, sem0)
            start_gather(1, rows1, sem1)

            @pl.loop(0, (cnt - 2) // 3)
            def _(k3):
                c0 = 3 * k3
                start_gather(c0 + 2, rows2, sem2)
                wait_gather(rows0, sem0)
                scatter(c0, rows0)
                start_gather(c0 + 3, rows0, sem0)
                wait_gather(rows1, sem1)
                scatter(c0 + 1, rows1)
                start_gather(c0 + 4, rows1, sem1)
                wait_gather(rows2, sem2)
                scatter(c0 + 2, rows2)

            wait_gather(rows0, sem0)
            scatter(cnt - 2, rows0)
            wait_gather(rows1, sem1)
            scatter(cnt - 1, rows1)

        for start, cnt in PHASES:
            do_phase(start, cnt)

        plsc.subcore_barrier()
        pltpu.sync_copy(
            acc.at[pl.ds(sid * RPT, RPT)],
            out_hbm.at[cid].at[pl.ds(sid * RPT, RPT)],
        )

        @pl.when(sid == NS - 1)
        def _():
            pltpu.sync_copy(
                acc.at[pl.ds(NS * RPT, TAIL)],
                out_hbm.at[cid].at[pl.ds(NS * RPT, TAIL)],
            )

    return k(h, edges4d, zeros)


def _bn_relu(z, g, b):
    mu = jnp.mean(z, axis=0, keepdims=True)
    zc = z - mu
    var = jnp.mean(zc * zc, axis=0, keepdims=True)
    z = zc * lax.rsqrt(var + BN_EPS) * g + b
    return jnp.maximum(z, 0.0)


def _tc_layer(h, p, W1, b1, g1, be1, W2, b2, gout, bout):
    def body(h_ref, p_ref, W1_ref, b1_ref, g1_ref, be1_ref, W2_ref, b2_ref,
             gout_ref, bout_ref, o_ref):
        z = h_ref[...] + p_ref[0] + p_ref[1]
        z = jnp.dot(z, W1_ref[...], preferred_element_type=jnp.float32) + b1_ref[...]
        z = _bn_relu(z, g1_ref[...], be1_ref[...])
        z = jnp.dot(z, W2_ref[...], preferred_element_type=jnp.float32) + b2_ref[...]
        o_ref[...] = _bn_relu(z, gout_ref[...], bout_ref[...])

    return pl.pallas_call(
        body,
        out_shape=jax.ShapeDtypeStruct((N, H), jnp.float32),
    )(h, p, W1, b1, g1, be1, W2, b2, gout, bout)


def _tc_layer_pool(h, p, W1, b1, g1, be1, W2, b2, gout, bout,
                   batch2d, pred_W, pred_b):
    def body(h_ref, p_ref, W1_ref, b1_ref, g1_ref, be1_ref, W2_ref, b2_ref,
             gout_ref, bout_ref, batch_ref, pW_ref, pb_ref, o_ref):
        z = h_ref[...] + p_ref[0] + p_ref[1]
        z = jnp.dot(z, W1_ref[...], preferred_element_type=jnp.float32) + b1_ref[...]
        z = _bn_relu(z, g1_ref[...], be1_ref[...])
        z = jnp.dot(z, W2_ref[...], preferred_element_type=jnp.float32) + b2_ref[...]
        h2 = _bn_relu(z, gout_ref[...], bout_ref[...])

        gids = lax.broadcasted_iota(jnp.int32, (G, N), 0)
        oh = (batch_ref[...] == gids).astype(jnp.float32)
        sums = jnp.dot(oh, h2, preferred_element_type=jnp.float32)
        counts = jnp.sum(oh, axis=1, keepdims=True)
        pooled = sums / jnp.maximum(counts, 1.0)
        out = jnp.dot(pooled, pW_ref[...], preferred_element_type=jnp.float32)
        out = out + pb_ref[...]
        m = jnp.max(out, axis=-1, keepdims=True)
        sh = out - m
        lse = jnp.log(jnp.sum(jnp.exp(sh), axis=-1, keepdims=True))
        o_ref[...] = sh - lse

    return pl.pallas_call(
        body,
        out_shape=jax.ShapeDtypeStruct((G, OUT), jnp.float32),
    )(h, p, W1, b1, g1, be1, W2, b2, gout, bout, batch2d, pred_W, pred_b)


def kernel(x, params, edge_index, batch):
    edges4d = edge_index.reshape(2, NW, NCHUNK, CHUNK)
    zeros = jnp.zeros((N, D), jnp.float32)
    batch2d = batch.reshape(1, N)

    l0, l1 = params["layers"]
    r = lambda v: v.reshape(1, -1)

    p1 = _sc_segment_sum(x, edges4d, zeros)
    h1 = _tc_layer(x, p1, l0["W1"], r(l0["b1"]), r(l0["g1"]), r(l0["be1"]),
                   l0["W2"], r(l0["b2"]), r(l0["gout"]), r(l0["bout"]))
    p2 = _sc_segment_sum(h1, edges4d, zeros)
    return _tc_layer_pool(h1, p2, l1["W1"], r(l1["b1"]), r(l1["g1"]),
                          r(l1["be1"]), l1["W2"], r(l1["b2"]), r(l1["gout"]),
                          r(l1["bout"]), batch2d, params["pred_W"],
                          r(params["pred_b"]))


# speedup vs baseline: 1.0679x; 1.0126x over previous
import functools

import jax
import jax.numpy as jnp
from jax import lax
from jax.experimental import pallas as pl
from jax.experimental.pallas import tpu as pltpu
from jax.experimental.pallas import tpu_sc as plsc

N = 10000
E = 320000
D = 128
H = 128
OUT = 64
G = 64
BN_EPS = 1e-5

NC = 2
NS = 16
NW = NC * NS
CHUNK = 80
EPW = E // NW
NCHUNK = EPW // CHUNK
PHASES = ((0, 32), (32, 32), (64, 32), (96, 29))
STAGE = 32
RPT = 624
TAIL = N - NS * RPT


def _sc_segment_sum(h, edges4d):
    mesh = plsc.VectorSubcoreMesh(core_axis_name="c", subcore_axis_name="s")

    @functools.partial(
        pl.kernel,
        out_type=jax.ShapeDtypeStruct((NC, N, D), jnp.float32),
        mesh=mesh,
        scratch_types=[
            pltpu.VMEM((STAGE, CHUNK), jnp.int32),
            pltpu.VMEM((STAGE, CHUNK), jnp.int32),
            pltpu.VMEM((CHUNK, D), jnp.float32),
            pltpu.VMEM((CHUNK, D), jnp.float32),
            pltpu.VMEM((CHUNK, D), jnp.float32),
            pltpu.VMEM_SHARED((N, D), jnp.float32),
            pltpu.SemaphoreType.DMA,
            pltpu.SemaphoreType.DMA,
            pltpu.SemaphoreType.DMA,
        ],
    )
    def k(h_hbm, e_hbm, out_hbm,
          sidx, didx, rows0, rows1, rows2, acc, sem0, sem1, sem2):
        cid = lax.axis_index("c")
        sid = lax.axis_index("s")
        wid = cid * NS + sid
        src_hbm = e_hbm.at[0]
        dst_hbm = e_hbm.at[1]

        zv = jnp.zeros((16,), jnp.float32)

        @pl.loop(0, CHUNK)
        def _(i):
            for j in range(D // 16):
                rows2.at[i, pl.ds(j * 16, 16)][...] = zv

        for j in range(RPT // CHUNK):
            pltpu.async_copy(
                rows2, acc.at[pl.ds(sid * RPT + j * CHUNK, CHUNK)], sem2)
        _rem = RPT - (RPT // CHUNK) * CHUNK
        pltpu.async_copy(
            rows2.at[pl.ds(0, _rem)],
            acc.at[pl.ds(sid * RPT + RPT - _rem, _rem)], sem2)

        @pl.when(sid == NS - 1)
        def _():
            pltpu.async_copy(
                rows2.at[pl.ds(0, TAIL)],
                acc.at[pl.ds(NS * RPT, TAIL)], sem2)

        def _drain_init():
            for j in range(RPT // CHUNK):
                pltpu.make_async_copy(
                    rows2, acc.at[pl.ds(sid * RPT + j * CHUNK, CHUNK)],
                    sem2).wait()
            pltpu.make_async_copy(
                rows2.at[pl.ds(0, _rem)],
                acc.at[pl.ds(sid * RPT + RPT - _rem, _rem)], sem2).wait()

            @pl.when(sid == NS - 1)
            def _():
                pltpu.make_async_copy(
                    rows2.at[pl.ds(0, TAIL)],
                    acc.at[pl.ds(NS * RPT, TAIL)], sem2).wait()

        def start_gather(c, buf, sem):
            pltpu.async_copy(h_hbm.at[sidx.at[c]], buf, sem)

        def wait_gather(buf, sem):
            pltpu.make_async_copy(h_hbm.at[sidx.at[0]], buf, sem).wait()

        def scatter(c, buf):
            pltpu.sync_copy(buf, acc.at[didx.at[c]], add=True)

        def do_phase(start, cnt):
            assert cnt % 3 == 2 and cnt >= 2
            pltpu.sync_copy(src_hbm.at[wid].at[pl.ds(start, cnt)],
                            sidx.at[pl.ds(0, cnt)])
            pltpu.sync_copy(dst_hbm.at[wid].at[pl.ds(start, cnt)],
                            didx.at[pl.ds(0, cnt)])
            if start == 0:
                _drain_init()
                plsc.subcore_barrier()
            start_gather(0, rows0, sem0)
            start_gather(1, rows1, sem1)

            @pl.loop(0, (cnt - 2) // 3)
            def _(k3):
                c0 = 3 * k3
                start_gather(c0 + 2, rows2, sem2)
                wait_gather(rows0, sem0)
                scatter(c0, rows0)
                start_gather(c0 + 3, rows0, sem0)
                wait_gather(rows1, sem1)
                scatter(c0 + 1, rows1)
                start_gather(c0 + 4, rows1, sem1)
                wait_gather(rows2, sem2)
                scatter(c0 + 2, rows2)

            wait_gather(rows0, sem0)
            scatter(cnt - 2, rows0)
            wait_gather(rows1, sem1)
            scatter(cnt - 1, rows1)

        for start, cnt in PHASES:
            do_phase(start, cnt)

        plsc.subcore_barrier()
        pltpu.sync_copy(
            acc.at[pl.ds(sid * RPT, RPT)],
            out_hbm.at[cid].at[pl.ds(sid * RPT, RPT)],
        )

        @pl.when(sid == NS - 1)
        def _():
            pltpu.sync_copy(
                acc.at[pl.ds(NS * RPT, TAIL)],
                out_hbm.at[cid].at[pl.ds(NS * RPT, TAIL)],
            )

    return k(h, edges4d)


def _mm(a, b):
    return jnp.dot(a.astype(jnp.bfloat16), b.astype(jnp.bfloat16),
                   preferred_element_type=jnp.float32)


def _bn_relu(z, g, b):
    mu = jnp.mean(z, axis=0, keepdims=True)
    zc = z - mu
    var = jnp.mean(zc * zc, axis=0, keepdims=True)
    z = zc * lax.rsqrt(var + BN_EPS) * g + b
    return jnp.maximum(z, 0.0)


def _tc_layer(h, p, W1, b1, g1, be1, W2, b2, gout, bout):
    def body(h_ref, p_ref, W1_ref, b1_ref, g1_ref, be1_ref, W2_ref, b2_ref,
             gout_ref, bout_ref, o_ref):
        z = h_ref[...] + p_ref[0] + p_ref[1]
        z = _mm(z, W1_ref[...]) + b1_ref[...]
        z = _bn_relu(z, g1_ref[...], be1_ref[...])
        z = _mm(z, W2_ref[...]) + b2_ref[...]
        o_ref[...] = _bn_relu(z, gout_ref[...], bout_ref[...])

    return pl.pallas_call(
        body,
        out_shape=jax.ShapeDtypeStruct((N, H), jnp.float32),
    )(h, p, W1, b1, g1, be1, W2, b2, gout, bout)


def _tc_layer_pool(h, p, W1, b1, g1, be1, W2, b2, gout, bout,
                   batch2d, pred_W, pred_b):
    def body(h_ref, p_ref, W1_ref, b1_ref, g1_ref, be1_ref, W2_ref, b2_ref,
             gout_ref, bout_ref, batch_ref, pW_ref, pb_ref, o_ref):
        z = h_ref[...] + p_ref[0] + p_ref[1]
        z = _mm(z, W1_ref[...]) + b1_ref[...]
        z = _bn_relu(z, g1_ref[...], be1_ref[...])
        z = _mm(z, W2_ref[...]) + b2_ref[...]
        h2 = _bn_relu(z, gout_ref[...], bout_ref[...])

        gids = lax.broadcasted_iota(jnp.int32, (G, N), 0)
        oh = (batch_ref[...] == gids).astype(jnp.float32)
        sums = _mm(oh, h2)
        counts = jnp.sum(oh, axis=1, keepdims=True)
        pooled = sums / jnp.maximum(counts, 1.0)
        out = _mm(pooled, pW_ref[...])
        out = out + pb_ref[...]
        m = jnp.max(out, axis=-1, keepdims=True)
        sh = out - m
        lse = jnp.log(jnp.sum(jnp.exp(sh), axis=-1, keepdims=True))
        o_ref[...] = sh - lse

    return pl.pallas_call(
        body,
        out_shape=jax.ShapeDtypeStruct((G, OUT), jnp.float32),
    )(h, p, W1, b1, g1, be1, W2, b2, gout, bout, batch2d, pred_W, pred_b)


def kernel(x, params, edge_index, batch):
    edges4d = edge_index.reshape(2, NW, NCHUNK, CHUNK)
    batch2d = batch.reshape(1, N)

    l0, l1 = params["layers"]
    r = lambda v: v.reshape(1, -1)

    p1 = _sc_segment_sum(x, edges4d)
    h1 = _tc_layer(x, p1, l0["W1"], r(l0["b1"]), r(l0["g1"]), r(l0["be1"]),
                   l0["W2"], r(l0["b2"]), r(l0["gout"]), r(l0["bout"]))
    p2 = _sc_segment_sum(h1, edges4d)
    return _tc_layer_pool(h1, p2, l1["W1"], r(l1["b1"]), r(l1["g1"]),
                          r(l1["be1"]), l1["W2"], r(l1["b2"]), r(l1["gout"]),
                          r(l1["bout"]), batch2d, params["pred_W"],
                          r(params["pred_b"]))
